# baseline (speedup 1.0000x reference)
import functools

import jax
import jax.numpy as jnp
from jax import lax
from jax.experimental import pallas as pl
from jax.experimental.pallas import tpu as pltpu
from jax.experimental.pallas import tpu_sc as plsc

_interpret = False

MIC = 2048
DIS = 1024
GT_HEAD = 4
GAT_HEADS = 10


def _pick_tile(n, cands):
    for c in cands:
        if n % c == 0:
            return c
    return n


def _mm_body(nres, x_ref, w_ref, *refs):
    out_ref = refs[-1]
    acc = jnp.dot(x_ref[...].astype(jnp.bfloat16),
                  w_ref[...].astype(jnp.bfloat16),
                  preferred_element_type=jnp.float32)
    for r in refs[:nres]:
        acc = acc + r[...]
    out_ref[...] = acc


def _matmul(x, w, residuals=()):
    M, K = x.shape
    _, N = w.shape
    tm = _pick_tile(M, (256, 128, 64))
    tn = _pick_tile(N, (512, 256, 128, 64))
    grid = (N // tn, M // tm)
    in_specs = [
        pl.BlockSpec((tm, K), lambda j, i: (i, 0)),
        pl.BlockSpec((K, tn), lambda j, i: (0, j)),
    ] + [pl.BlockSpec((tm, tn), lambda j, i: (i, j)) for _ in residuals]
    return pl.pallas_call(
        functools.partial(_mm_body, len(residuals)),
        grid=grid,
        in_specs=in_specs,
        out_specs=pl.BlockSpec((tm, tn), lambda j, i: (i, j)),
        out_shape=jax.ShapeDtypeStruct((M, N), jnp.float32),
        interpret=_interpret,
    )(x, w, *residuals)


def _lnc_body(c_ref, o_ref):
    c = c_ref[...]
    o_ref[...] = jnp.where(c > 0.0, jnp.log(c), -jnp.inf)


def _ln_counts(counts, n):
    tr = 256
    return pl.pallas_call(
        _lnc_body,
        grid=(n // tr,),
        in_specs=[pl.BlockSpec((tr, n), lambda i: (i, 0))],
        out_specs=pl.BlockSpec((tr, n), lambda i: (i, 0)),
        out_shape=jax.ShapeDtypeStruct((n, n), jnp.float32),
        interpret=_interpret,
    )(counts)


def _masked_softmax_parts(s):
    m = jnp.max(s, axis=1, keepdims=True)
    mf = jnp.maximum(m, -1e30)
    e = jnp.exp(s - mf)
    return e, 1.0 / (jnp.sum(e, axis=1, keepdims=True) + 1e-9)


def _gt_attn_body(dh, q_ref, k_ref, v_ref, lnc_ref, o_ref):
    lnc = lnc_ref[...]
    outs = []
    for h in range(GT_HEAD):
        sl = slice(h * dh, (h + 1) * dh)
        s = lax.dot_general(q_ref[:, sl].astype(jnp.bfloat16),
                            k_ref[:, sl].astype(jnp.bfloat16),
                            (((1,), (1,)), ((), ())),
                            preferred_element_type=jnp.float32) + lnc
        e, inv = _masked_softmax_parts(s)
        outs.append(jnp.dot(e.astype(jnp.bfloat16),
                            v_ref[:, sl].astype(jnp.bfloat16),
                            preferred_element_type=jnp.float32) * inv)
    o_ref[...] = jnp.concatenate(outs, axis=1)


def _gt_attention(qkv, lnc, n, d):
    dh = d // GT_HEAD
    td = 256
    grid = (n // td,)
    return pl.pallas_call(
        functools.partial(_gt_attn_body, dh),
        grid=grid,
        in_specs=[
            pl.BlockSpec((td, d), lambda i: (i, 0)),
            pl.BlockSpec((n, d), lambda i: (0, 1)),
            pl.BlockSpec((n, d), lambda i: (0, 2)),
            pl.BlockSpec((td, n), lambda i: (i, 0)),
        ],
        out_specs=pl.BlockSpec((td, d), lambda i: (i, 0)),
        out_shape=jax.ShapeDtypeStruct((n, d), jnp.float32),
        interpret=_interpret,
    )(qkv, qkv, qkv, lnc)


def _gat_attn_body(h, whd_ref, whs_ref, al_ref, ar_ref, lnc_ref, o_ref):
    whd = whd_ref[...]
    whs = whs_ref[...]
    al = al_ref[...]
    ar = ar_ref[...]
    lnc = lnc_ref[...]
    outs = []
    for t in range(GAT_HEADS):
        sl = slice(t * h, (t + 1) * h)
        whd_t = whd[:, sl]
        whs_t = whs[:, sl]
        ed = lax.dot_general(whd_t, al[t:t + 1, :], (((1,), (1,)), ((), ())),
                             preferred_element_type=jnp.float32)
        es = lax.dot_general(ar[t:t + 1, :], whs_t, (((1,), (1,)), ((), ())),
                             preferred_element_type=jnp.float32)
        s = ed + es
        s = jnp.where(s >= 0, s, 0.2 * s) + lnc
        e, inv = _masked_softmax_parts(s)
        out = jnp.dot(e.astype(jnp.bfloat16), whs_t.astype(jnp.bfloat16),
                      preferred_element_type=jnp.float32) * inv
        outs.append(jnp.where(out > 0, out, jnp.exp(out) - 1.0))
    o_ref[...] = jnp.concatenate(outs, axis=1)


def _gat_attention(wh, al, ar, lnc, n, h):
    td = 256
    grid = (n // td,)
    return pl.pallas_call(
        functools.partial(_gat_attn_body, h),
        grid=grid,
        in_specs=[
            pl.BlockSpec((td, GAT_HEADS * h), lambda i: (i, 0)),
            pl.BlockSpec((n, GAT_HEADS * h), lambda i: (0, 0)),
            pl.BlockSpec((GAT_HEADS, h), lambda i: (0, 0)),
            pl.BlockSpec((GAT_HEADS, h), lambda i: (0, 0)),
            pl.BlockSpec((td, n), lambda i: (i, 0)),
        ],
        out_specs=pl.BlockSpec((td, GAT_HEADS * h), lambda i: (i, 0)),
        out_shape=jax.ShapeDtypeStruct((n, GAT_HEADS * h), jnp.float32),
        interpret=_interpret,
    )(wh, wh, al, ar, lnc)


def _mlp_body(gm_ref, gma_ref, gd_ref, gda_ref, w1t_ref, w1b_ref, b1_ref,
              w2_ref, b2_ref, o_ref):
    hm = jnp.dot(gm_ref[...] + gma_ref[...], w1t_ref[...],
                 preferred_element_type=jnp.float32)
    hd = jnp.dot(gd_ref[...] + gda_ref[...], w1b_ref[...],
                 preferred_element_type=jnp.float32)
    h = hm + hd + b1_ref[...]
    h = jnp.where(h > 0, h, jnp.exp(h) - 1.0)
    r = jnp.dot(h, w2_ref[...], preferred_element_type=jnp.float32) + b2_ref[...]
    o_ref[...] = 1.0 / (1.0 + jnp.exp(-r))


def _mlp_head(gm, gma, gd, gda, w1, b1, w2, b2):
    b = gm.shape[0]
    tb = 512
    w1t = w1[:64]
    w1b = w1[64:]
    w2p = jnp.zeros((64, 128), jnp.float32).at[:, :1].set(w2)
    b2p = jnp.zeros((1, 128), jnp.float32).at[0, 0].set(b2[0])
    grid = (b // tb,)
    out = pl.pallas_call(
        _mlp_body,
        grid=grid,
        in_specs=[
            pl.BlockSpec((tb, 64), lambda i: (i, 0)),
            pl.BlockSpec((tb, 64), lambda i: (i, 0)),
            pl.BlockSpec((tb, 64), lambda i: (i, 0)),
            pl.BlockSpec((tb, 64), lambda i: (i, 0)),
            pl.BlockSpec((64, 64), lambda i: (0, 0)),
            pl.BlockSpec((64, 64), lambda i: (0, 0)),
            pl.BlockSpec((1, 64), lambda i: (0, 0)),
            pl.BlockSpec((64, 128), lambda i: (0, 0)),
            pl.BlockSpec((1, 128), lambda i: (0, 0)),
        ],
        out_specs=pl.BlockSpec((tb, 128), lambda i: (i, 0)),
        out_shape=jax.ShapeDtypeStruct((b, 128), jnp.float32),
        interpret=_interpret,
    )(gm, gma, gd, gda, w1t, w1b, b1.reshape(1, 64), w2p, b2p)
    return out[:, :1]


def _sc_build_counts(edge_index, n_nodes, n_chunk):
    src = edge_index[0].astype(jnp.int32)
    dst = edge_index[1].astype(jnp.int32)
    e = src.shape[0]
    info = plsc.get_sparse_core_info()
    nc, ns = info.num_cores, info.num_subcores
    ept = e // ns
    nj = ept // 128
    n_chunks = n_nodes // n_chunk
    rpc = n_chunk * n_nodes // 16
    zrows = rpc // ns
    dump = rpc

    zeros_sp = jnp.zeros((zrows, 16), jnp.float32)
    payload = (src[:, None] % 16 == lax.iota(jnp.int32, 16)[None, :]
               ).astype(jnp.float32)
    mesh = plsc.VectorSubcoreMesh(core_axis_name="c", subcore_axis_name="s")

    @functools.partial(
        pl.kernel, mesh=mesh,
        compiler_params=pltpu.CompilerParams(use_tc_tiling_on_sc=False),
        out_type=jax.ShapeDtypeStruct((n_nodes * n_nodes // 16, 16), jnp.float32),
        scratch_types=[
            pltpu.VMEM((ept,), jnp.int32),
            pltpu.VMEM((ept,), jnp.int32),
            pltpu.VMEM((ept, 16), jnp.float32),
            pltpu.VMEM((ept,), jnp.int32),
            pltpu.VMEM_SHARED((rpc + 8, 16), jnp.float32),
        ],
    )
    def k(src_h, dst_h, zsp_h, pay_h, out_h, src_v, dst_v, pay, ridx, acc):
        cid = lax.axis_index("c")
        sid = lax.axis_index("s")
        base = sid * ept
        pltpu.sync_copy(src_h.at[pl.ds(base, ept)], src_v)
        pltpu.sync_copy(dst_h.at[pl.ds(base, ept)], dst_v)
        pltpu.sync_copy(pay_h.at[pl.ds(base, ept)], pay)

        for c in range(n_chunks):
            @pl.when(cid == (c % nc))
            def _():
                pltpu.sync_copy(zsp_h, acc.at[pl.ds(sid * zrows, zrows)])
                plsc.subcore_barrier()

                def idx_body(g, carry):
                    sv = src_v[pl.ds(g * 16, 16)]
                    dv = dst_v[pl.ds(g * 16, 16)]
                    rel = dv - (c * n_chunk)
                    inb = jnp.logical_and(rel >= 0, rel < n_chunk)
                    row = rel * (n_nodes // 16) + lax.shift_right_logical(sv, 4)
                    row = jnp.where(inb, row, dump)
                    ridx[pl.ds(g * 16, 16)] = row
                    return carry

                lax.fori_loop(0, ept // 16, idx_body, 0)

                for r in range(ns):
                    @pl.when(sid == r)
                    def _():
                        pltpu.sync_copy(pay, acc.at[ridx], add=True)
                    plsc.subcore_barrier()
                pltpu.sync_copy(acc.at[pl.ds(sid * zrows, zrows)],
                                out_h.at[pl.ds(c * rpc + sid * zrows, zrows)])

    out = k(src, dst, zeros_sp, payload)
    return out.reshape(n_nodes, n_nodes)


def _sc_gather_embeddings(emb_m, emb_mm_ass, emb_d, emb_dd_ass, idx0, idx1):
    b = idx0.shape[0]
    info = plsc.get_sparse_core_info()
    nc, ns = info.num_cores, info.num_subcores
    bpw = b // (nc * ns)
    mesh = plsc.VectorSubcoreMesh(core_axis_name="c", subcore_axis_name="s")

    @functools.partial(
        pl.kernel, mesh=mesh,
        compiler_params=pltpu.CompilerParams(use_tc_tiling_on_sc=False),
        out_type=[jax.ShapeDtypeStruct((b, 64), jnp.float32)] * 4,
        scratch_types=[
            pltpu.VMEM((bpw,), jnp.int32),
            pltpu.VMEM((bpw,), jnp.int32),
            pltpu.VMEM((bpw, 64), jnp.float32),
            pltpu.SemaphoreType.DMA,
        ],
    )
    def k(em, ema, ed, eda, i0, i1, o0, o1, o2, o3, iv0, iv1, rows, sem):
        wid = lax.axis_index("s") * nc + lax.axis_index("c")
        base = wid * bpw
        pltpu.sync_copy(i0.at[pl.ds(base, bpw)], iv0)
        pltpu.sync_copy(i1.at[pl.ds(base, bpw)], iv1)
        for table, iv, out in ((em, iv0, o0), (ema, iv0, o1),
                               (ed, iv1, o2), (eda, iv1, o3)):
            pltpu.async_copy(table.at[iv], rows, sem).wait()
            pltpu.sync_copy(rows, out.at[pl.ds(base, bpw)])

    return k(emb_m, emb_mm_ass, emb_d, emb_dd_ass, idx0, idx1)


def _gt_block(x, counts, layers, extra_res):
    n, d = x.shape
    scale = 1.0 / ((d // GT_HEAD) ** 0.5)
    for li, lp in enumerate(layers):
        wqkv = jnp.concatenate([lp['Wq'] * scale, lp['Wk'], lp['Wv']], axis=1)
        qkv = _matmul(x, wqkv)
        agg = _gt_attention(qkv, counts, n, d)
        res = (x,) if (li < len(layers) - 1 or extra_res is None) else (x, extra_res)
        x = _matmul(agg, lp['Wo'], residuals=res)
    return x


def _gat_block(x, counts, p):
    n = x.shape[0]
    for lp in p['layers']:
        h = lp['al'].shape[-1]
        wh = _matmul(x, lp['W'])
        x = _gat_attention(wh, lp['al'], lp['ar'], counts, n, h)
    return _matmul(x, p['Wout'])


def kernel(microe, disease, params, mm_graph, dd_graph, md_graph, samples, epoch):
    c_mm = _ln_counts(_sc_build_counts(mm_graph, MIC, 512), MIC)
    c_dd = _ln_counts(_sc_build_counts(dd_graph, DIS, 512), DIS)
    c_md = _ln_counts(_sc_build_counts(md_graph, MIC + DIS, 384), MIC + DIS)

    xm = _gt_block(microe, c_mm, params['gt_m'], extra_res=microe)
    xd = _gt_block(disease, c_dd, params['gt_d'], extra_res=disease)

    emb_m = _gat_block(xm, c_mm, params['gat_m'])
    emb_d = _gat_block(xd, c_dd, params['gat_d'])

    xmd_top = _matmul(microe, 2.0 * params['lin_m'])
    xmd_bot = _matmul(disease, 2.0 * params['lin_d'])
    xmd = jnp.concatenate([xmd_top, xmd_bot], axis=0)
    emb_md = _gat_block(xmd, c_md, params['gat_md'])
    emb_mm_ass = emb_md[:MIC]
    emb_dd_ass = emb_md[MIC:]

    idx0 = samples[:, 0].astype(jnp.int32)
    idx1 = samples[:, 1].astype(jnp.int32)
    gm, gma, gd, gda = _sc_gather_embeddings(emb_m, emb_mm_ass, emb_d,
                                             emb_dd_ass, idx0, idx1)

    mlp = params['mlp']
    result = _mlp_head(gm, gma, gd, gda, mlp['W1'], mlp['b1'], mlp['W2'], mlp['b2'])
    return (result, emb_m, emb_mm_ass, emb_d, emb_dd_ass)


# speedup vs baseline: 1.0239x; 1.0239x over previous
import functools

import jax
import jax.numpy as jnp
from jax import lax
from jax.experimental import pallas as pl
from jax.experimental.pallas import tpu as pltpu
from jax.experimental.pallas import tpu_sc as plsc

_interpret = False

MIC = 2048
DIS = 1024
GT_HEAD = 4
GAT_HEADS = 10


def _pick_tile(n, cands):
    for c in cands:
        if n % c == 0:
            return c
    return n


def _mm_body(nres, x_ref, w_ref, *refs):
    out_ref = refs[-1]
    acc = jnp.dot(x_ref[...], w_ref[...], preferred_element_type=jnp.float32)
    for r in refs[:nres]:
        acc = acc + r[...]
    out_ref[...] = acc


def _matmul(x, w, residuals=()):
    M, K = x.shape
    _, N = w.shape
    tm = _pick_tile(M, (256, 128, 64))
    tn = _pick_tile(N, (512, 256, 128, 64))
    grid = (N // tn, M // tm)
    in_specs = [
        pl.BlockSpec((tm, K), lambda j, i: (i, 0)),
        pl.BlockSpec((K, tn), lambda j, i: (0, j)),
    ] + [pl.BlockSpec((tm, tn), lambda j, i: (i, j)) for _ in residuals]
    return pl.pallas_call(
        functools.partial(_mm_body, len(residuals)),
        grid=grid,
        in_specs=in_specs,
        out_specs=pl.BlockSpec((tm, tn), lambda j, i: (i, j)),
        out_shape=jax.ShapeDtypeStruct((M, N), jnp.float32),
        interpret=_interpret,
    )(x, w, *residuals)


def _lnc_body(c_ref, o_ref):
    c = c_ref[...]
    o_ref[...] = jnp.where(c > 0.0, jnp.log(c), -jnp.inf)


def _ln_counts(counts, n):
    tr = 256
    return pl.pallas_call(
        _lnc_body,
        grid=(n // tr,),
        in_specs=[pl.BlockSpec((tr, n), lambda i: (i, 0))],
        out_specs=pl.BlockSpec((tr, n), lambda i: (i, 0)),
        out_shape=jax.ShapeDtypeStruct((n, n), jnp.float32),
        interpret=_interpret,
    )(counts)


def _masked_softmax_parts(s):
    m = jnp.max(s, axis=1, keepdims=True)
    mf = jnp.maximum(m, -1e30)
    e = jnp.exp(s - mf)
    return e, 1.0 / (jnp.sum(e, axis=1, keepdims=True) + 1e-9)


def _gt_attn_body(dh, q_ref, k_ref, v_ref, lnc_ref, o_ref):
    lnc = lnc_ref[...]
    outs = []
    for h in range(GT_HEAD):
        sl = slice(h * dh, (h + 1) * dh)
        s = lax.dot_general(q_ref[:, sl], k_ref[:, sl], (((1,), (1,)), ((), ())),
                            preferred_element_type=jnp.float32) + lnc
        e, inv = _masked_softmax_parts(s)
        outs.append(jnp.dot(e, v_ref[:, sl],
                            preferred_element_type=jnp.float32) * inv)
    o_ref[...] = jnp.concatenate(outs, axis=1)


def _gt_attention(qkv, lnc, n, d):
    dh = d // GT_HEAD
    td = 256
    grid = (n // td,)
    return pl.pallas_call(
        functools.partial(_gt_attn_body, dh),
        grid=grid,
        in_specs=[
            pl.BlockSpec((td, d), lambda i: (i, 0)),
            pl.BlockSpec((n, d), lambda i: (0, 1)),
            pl.BlockSpec((n, d), lambda i: (0, 2)),
            pl.BlockSpec((td, n), lambda i: (i, 0)),
        ],
        out_specs=pl.BlockSpec((td, d), lambda i: (i, 0)),
        out_shape=jax.ShapeDtypeStruct((n, d), jnp.float32),
        interpret=_interpret,
    )(qkv, qkv, qkv, lnc)


def _gat_attn_body(h, whd_ref, whs_ref, al_ref, ar_ref, lnc_ref, o_ref):
    whd = whd_ref[...]
    whs = whs_ref[...]
    al = al_ref[...]
    ar = ar_ref[...]
    lnc = lnc_ref[...]
    outs = []
    for t in range(GAT_HEADS):
        sl = slice(t * h, (t + 1) * h)
        whd_t = whd[:, sl]
        whs_t = whs[:, sl]
        ed = lax.dot_general(whd_t, al[t:t + 1, :], (((1,), (1,)), ((), ())),
                             preferred_element_type=jnp.float32)
        es = lax.dot_general(ar[t:t + 1, :], whs_t, (((1,), (1,)), ((), ())),
                             preferred_element_type=jnp.float32)
        s = ed + es
        s = jnp.where(s >= 0, s, 0.2 * s) + lnc
        e, inv = _masked_softmax_parts(s)
        out = jnp.dot(e, whs_t, preferred_element_type=jnp.float32) * inv
        outs.append(jnp.where(out > 0, out, jnp.exp(out) - 1.0))
    o_ref[...] = jnp.concatenate(outs, axis=1)


def _gat_attention(wh, al, ar, lnc, n, h):
    td = 256
    grid = (n // td,)
    return pl.pallas_call(
        functools.partial(_gat_attn_body, h),
        grid=grid,
        in_specs=[
            pl.BlockSpec((td, GAT_HEADS * h), lambda i: (i, 0)),
            pl.BlockSpec((n, GAT_HEADS * h), lambda i: (0, 0)),
            pl.BlockSpec((GAT_HEADS, h), lambda i: (0, 0)),
            pl.BlockSpec((GAT_HEADS, h), lambda i: (0, 0)),
            pl.BlockSpec((td, n), lambda i: (i, 0)),
        ],
        out_specs=pl.BlockSpec((td, GAT_HEADS * h), lambda i: (i, 0)),
        out_shape=jax.ShapeDtypeStruct((n, GAT_HEADS * h), jnp.float32),
        interpret=_interpret,
    )(wh, wh, al, ar, lnc)


def _mlp_body(gm_ref, gma_ref, gd_ref, gda_ref, w1t_ref, w1b_ref, b1_ref,
              w2_ref, b2_ref, o_ref):
    hm = jnp.dot(gm_ref[...] + gma_ref[...], w1t_ref[...],
                 preferred_element_type=jnp.float32)
    hd = jnp.dot(gd_ref[...] + gda_ref[...], w1b_ref[...],
                 preferred_element_type=jnp.float32)
    h = hm + hd + b1_ref[...]
    h = jnp.where(h > 0, h, jnp.exp(h) - 1.0)
    r = jnp.dot(h, w2_ref[...], preferred_element_type=jnp.float32) + b2_ref[...]
    o_ref[...] = 1.0 / (1.0 + jnp.exp(-r))


def _mlp_head(gm, gma, gd, gda, w1, b1, w2, b2):
    b = gm.shape[0]
    tb = 512
    w1t = w1[:64]
    w1b = w1[64:]
    w2p = jnp.zeros((64, 128), jnp.float32).at[:, :1].set(w2)
    b2p = jnp.zeros((1, 128), jnp.float32).at[0, 0].set(b2[0])
    grid = (b // tb,)
    out = pl.pallas_call(
        _mlp_body,
        grid=grid,
        in_specs=[
            pl.BlockSpec((tb, 64), lambda i: (i, 0)),
            pl.BlockSpec((tb, 64), lambda i: (i, 0)),
            pl.BlockSpec((tb, 64), lambda i: (i, 0)),
            pl.BlockSpec((tb, 64), lambda i: (i, 0)),
            pl.BlockSpec((64, 64), lambda i: (0, 0)),
            pl.BlockSpec((64, 64), lambda i: (0, 0)),
            pl.BlockSpec((1, 64), lambda i: (0, 0)),
            pl.BlockSpec((64, 128), lambda i: (0, 0)),
            pl.BlockSpec((1, 128), lambda i: (0, 0)),
        ],
        out_specs=pl.BlockSpec((tb, 128), lambda i: (i, 0)),
        out_shape=jax.ShapeDtypeStruct((b, 128), jnp.float32),
        interpret=_interpret,
    )(gm, gma, gd, gda, w1t, w1b, b1.reshape(1, 64), w2p, b2p)
    return out[:, :1]


def _sc_build_counts(edge_index, n_nodes, n_chunk):
    src = edge_index[0].astype(jnp.int32)
    dst = edge_index[1].astype(jnp.int32)
    e = src.shape[0]
    info = plsc.get_sparse_core_info()
    nc, ns = info.num_cores, info.num_subcores
    ept = e // ns
    nj = ept // 128
    n_chunks = n_nodes // n_chunk
    rpc = n_chunk * n_nodes // 16
    zrows = rpc // ns
    dump = rpc

    zeros_sp = jnp.zeros((zrows, 16), jnp.float32)
    payload = (src[:, None] % 16 == lax.iota(jnp.int32, 16)[None, :]
               ).astype(jnp.float32)
    mesh = plsc.VectorSubcoreMesh(core_axis_name="c", subcore_axis_name="s")

    @functools.partial(
        pl.kernel, mesh=mesh,
        compiler_params=pltpu.CompilerParams(use_tc_tiling_on_sc=False),
        out_type=jax.ShapeDtypeStruct((n_nodes * n_nodes // 16, 16), jnp.float32),
        scratch_types=[
            pltpu.VMEM((ept,), jnp.int32),
            pltpu.VMEM((ept,), jnp.int32),
            pltpu.VMEM((ept, 16), jnp.float32),
            pltpu.VMEM((ept,), jnp.int32),
            pltpu.VMEM_SHARED((rpc + 8, 16), jnp.float32),
        ],
    )
    def k(src_h, dst_h, zsp_h, pay_h, out_h, src_v, dst_v, pay, ridx, acc):
        cid = lax.axis_index("c")
        sid = lax.axis_index("s")
        base = sid * ept
        pltpu.sync_copy(src_h.at[pl.ds(base, ept)], src_v)
        pltpu.sync_copy(dst_h.at[pl.ds(base, ept)], dst_v)
        pltpu.sync_copy(pay_h.at[pl.ds(base, ept)], pay)

        for c in range(n_chunks):
            @pl.when(cid == (c % nc))
            def _():
                pltpu.sync_copy(zsp_h, acc.at[pl.ds(sid * zrows, zrows)])
                plsc.subcore_barrier()

                def idx_body(g, carry):
                    sv = src_v[pl.ds(g * 16, 16)]
                    dv = dst_v[pl.ds(g * 16, 16)]
                    rel = dv - (c * n_chunk)
                    inb = jnp.logical_and(rel >= 0, rel < n_chunk)
                    row = rel * (n_nodes // 16) + lax.shift_right_logical(sv, 4)
                    row = jnp.where(inb, row, dump)
                    ridx[pl.ds(g * 16, 16)] = row
                    return carry

                lax.fori_loop(0, ept // 16, idx_body, 0)

                for r in range(ns):
                    @pl.when(sid == r)
                    def _():
                        pltpu.sync_copy(pay, acc.at[ridx], add=True)
                    plsc.subcore_barrier()
                pltpu.sync_copy(acc.at[pl.ds(sid * zrows, zrows)],
                                out_h.at[pl.ds(c * rpc + sid * zrows, zrows)])

    out = k(src, dst, zeros_sp, payload)
    return out.reshape(n_nodes, n_nodes)


def _sc_gather_embeddings(emb_m, emb_mm_ass, emb_d, emb_dd_ass, idx0, idx1):
    b = idx0.shape[0]
    info = plsc.get_sparse_core_info()
    nc, ns = info.num_cores, info.num_subcores
    bpw = b // (nc * ns)
    mesh = plsc.VectorSubcoreMesh(core_axis_name="c", subcore_axis_name="s")

    @functools.partial(
        pl.kernel, mesh=mesh,
        compiler_params=pltpu.CompilerParams(use_tc_tiling_on_sc=False),
        out_type=[jax.ShapeDtypeStruct((b, 64), jnp.float32)] * 4,
        scratch_types=[
            pltpu.VMEM((bpw,), jnp.int32),
            pltpu.VMEM((bpw,), jnp.int32),
            pltpu.VMEM((bpw, 64), jnp.float32),
            pltpu.SemaphoreType.DMA,
        ],
    )
    def k(em, ema, ed, eda, i0, i1, o0, o1, o2, o3, iv0, iv1, rows, sem):
        wid = lax.axis_index("s") * nc + lax.axis_index("c")
        base = wid * bpw
        pltpu.sync_copy(i0.at[pl.ds(base, bpw)], iv0)
        pltpu.sync_copy(i1.at[pl.ds(base, bpw)], iv1)
        for table, iv, out in ((em, iv0, o0), (ema, iv0, o1),
                               (ed, iv1, o2), (eda, iv1, o3)):
            pltpu.async_copy(table.at[iv], rows, sem).wait()
            pltpu.sync_copy(rows, out.at[pl.ds(base, bpw)])

    return k(emb_m, emb_mm_ass, emb_d, emb_dd_ass, idx0, idx1)


def _gt_block(x, counts, layers, extra_res):
    n, d = x.shape
    scale = 1.0 / ((d // GT_HEAD) ** 0.5)
    for li, lp in enumerate(layers):
        wqkv = jnp.concatenate([lp['Wq'] * scale, lp['Wk'], lp['Wv']], axis=1)
        qkv = _matmul(x, wqkv)
        agg = _gt_attention(qkv, counts, n, d)
        res = (x,) if (li < len(layers) - 1 or extra_res is None) else (x, extra_res)
        x = _matmul(agg, lp['Wo'], residuals=res)
    return x


def _gat_block(x, counts, p):
    n = x.shape[0]
    for lp in p['layers']:
        h = lp['al'].shape[-1]
        wh = _matmul(x, lp['W'])
        x = _gat_attention(wh, lp['al'], lp['ar'], counts, n, h)
    return _matmul(x, p['Wout'])


def kernel(microe, disease, params, mm_graph, dd_graph, md_graph, samples, epoch):
    c_mm = _ln_counts(_sc_build_counts(mm_graph, MIC, 512), MIC)
    c_dd = _ln_counts(_sc_build_counts(dd_graph, DIS, 512), DIS)
    c_md = _ln_counts(_sc_build_counts(md_graph, MIC + DIS, 384), MIC + DIS)

    xm = _gt_block(microe, c_mm, params['gt_m'], extra_res=microe)
    xd = _gt_block(disease, c_dd, params['gt_d'], extra_res=disease)

    emb_m = _gat_block(xm, c_mm, params['gat_m'])
    emb_d = _gat_block(xd, c_dd, params['gat_d'])

    xmd_top = _matmul(microe, 2.0 * params['lin_m'])
    xmd_bot = _matmul(disease, 2.0 * params['lin_d'])
    xmd = jnp.concatenate([xmd_top, xmd_bot], axis=0)
    emb_md = _gat_block(xmd, c_md, params['gat_md'])
    emb_mm_ass = emb_md[:MIC]
    emb_dd_ass = emb_md[MIC:]

    idx0 = samples[:, 0].astype(jnp.int32)
    idx1 = samples[:, 1].astype(jnp.int32)
    gm, gma, gd, gda = _sc_gather_embeddings(emb_m, emb_mm_ass, emb_d,
                                             emb_dd_ass, idx0, idx1)

    mlp = params['mlp']
    result = _mlp_head(gm, gma, gd, gda, mlp['W1'], mlp['b1'], mlp['W2'], mlp['b2'])
    return (result, emb_m, emb_mm_ass, emb_d, emb_dd_ass)


# speedup vs baseline: 1.0251x; 1.0011x over previous
import functools

import jax
import jax.numpy as jnp
from jax import lax
from jax.experimental import pallas as pl
from jax.experimental.pallas import tpu as pltpu
from jax.experimental.pallas import tpu_sc as plsc

_interpret = False

MIC = 2048
DIS = 1024
GT_HEAD = 4
GAT_HEADS = 10


def _pick_tile(n, cands):
    for c in cands:
        if n % c == 0:
            return c
    return n


def _mm_body(nres, x_ref, w_ref, *refs):
    out_ref = refs[-1]
    acc = jnp.dot(x_ref[...], w_ref[...], preferred_element_type=jnp.float32)
    for r in refs[:nres]:
        acc = acc + r[...]
    out_ref[...] = acc


def _matmul(x, w, residuals=()):
    M, K = x.shape
    _, N = w.shape
    tm = _pick_tile(M, (256, 128, 64))
    tn = _pick_tile(N, (512, 256, 128, 64))
    grid = (N // tn, M // tm)
    in_specs = [
        pl.BlockSpec((tm, K), lambda j, i: (i, 0)),
        pl.BlockSpec((K, tn), lambda j, i: (0, j)),
    ] + [pl.BlockSpec((tm, tn), lambda j, i: (i, j)) for _ in residuals]
    return pl.pallas_call(
        functools.partial(_mm_body, len(residuals)),
        grid=grid,
        in_specs=in_specs,
        out_specs=pl.BlockSpec((tm, tn), lambda j, i: (i, j)),
        out_shape=jax.ShapeDtypeStruct((M, N), jnp.float32),
        interpret=_interpret,
    )(x, w, *residuals)


def _lnc_body(c_ref, o_ref):
    c = c_ref[...]
    o_ref[...] = jnp.where(c > 0.0, jnp.log(c), -jnp.inf)


def _ln_counts(counts, n):
    tr = 256
    return pl.pallas_call(
        _lnc_body,
        grid=(n // tr,),
        in_specs=[pl.BlockSpec((tr, n), lambda i: (i, 0))],
        out_specs=pl.BlockSpec((tr, n), lambda i: (i, 0)),
        out_shape=jax.ShapeDtypeStruct((n, n), jnp.float32),
        interpret=_interpret,
    )(counts)


def _masked_softmax_parts(s):
    m = jnp.max(s, axis=1, keepdims=True)
    mf = jnp.maximum(m, -1e30)
    e = jnp.exp(s - mf)
    return e, 1.0 / (jnp.sum(e, axis=1, keepdims=True) + 1e-9)


def _gt_attn_body(dh, q_ref, k_ref, v_ref, lnc_ref, o_ref):
    lnc = lnc_ref[...]
    outs = []
    for h in range(GT_HEAD):
        sl = slice(h * dh, (h + 1) * dh)
        s = lax.dot_general(q_ref[:, sl], k_ref[:, sl], (((1,), (1,)), ((), ())),
                            preferred_element_type=jnp.float32) + lnc
        e, inv = _masked_softmax_parts(s)
        outs.append(jnp.dot(e, v_ref[:, sl],
                            preferred_element_type=jnp.float32) * inv)
    o_ref[...] = jnp.concatenate(outs, axis=1)


def _gt_attention(qkv, lnc, n, d):
    dh = d // GT_HEAD
    td = 256
    grid = (n // td,)
    return pl.pallas_call(
        functools.partial(_gt_attn_body, dh),
        grid=grid,
        in_specs=[
            pl.BlockSpec((td, d), lambda i: (i, 0)),
            pl.BlockSpec((n, d), lambda i: (0, 1)),
            pl.BlockSpec((n, d), lambda i: (0, 2)),
            pl.BlockSpec((td, n), lambda i: (i, 0)),
        ],
        out_specs=pl.BlockSpec((td, d), lambda i: (i, 0)),
        out_shape=jax.ShapeDtypeStruct((n, d), jnp.float32),
        interpret=_interpret,
    )(qkv, qkv, qkv, lnc)


def _gat_attn_body(h, whd_ref, whs_ref, al_ref, ar_ref, lnc_ref, o_ref):
    whd = whd_ref[...]
    whs = whs_ref[...]
    al = al_ref[...]
    ar = ar_ref[...]
    lnc = lnc_ref[...]
    outs = []
    for t in range(GAT_HEADS):
        sl = slice(t * h, (t + 1) * h)
        whd_t = whd[:, sl]
        whs_t = whs[:, sl]
        ed = lax.dot_general(whd_t, al[t:t + 1, :], (((1,), (1,)), ((), ())),
                             preferred_element_type=jnp.float32)
        es = lax.dot_general(ar[t:t + 1, :], whs_t, (((1,), (1,)), ((), ())),
                             preferred_element_type=jnp.float32)
        s = ed + es
        s = jnp.where(s >= 0, s, 0.2 * s) + lnc
        e, inv = _masked_softmax_parts(s)
        out = jnp.dot(e, whs_t, preferred_element_type=jnp.float32) * inv
        outs.append(jnp.where(out > 0, out, jnp.exp(out) - 1.0))
    o_ref[...] = jnp.concatenate(outs, axis=1)


def _gat_attention(wh, al, ar, lnc, n, h):
    td = 256
    grid = (n // td,)
    return pl.pallas_call(
        functools.partial(_gat_attn_body, h),
        grid=grid,
        in_specs=[
            pl.BlockSpec((td, GAT_HEADS * h), lambda i: (i, 0)),
            pl.BlockSpec((n, GAT_HEADS * h), lambda i: (0, 0)),
            pl.BlockSpec((GAT_HEADS, h), lambda i: (0, 0)),
            pl.BlockSpec((GAT_HEADS, h), lambda i: (0, 0)),
            pl.BlockSpec((td, n), lambda i: (i, 0)),
        ],
        out_specs=pl.BlockSpec((td, GAT_HEADS * h), lambda i: (i, 0)),
        out_shape=jax.ShapeDtypeStruct((n, GAT_HEADS * h), jnp.float32),
        interpret=_interpret,
    )(wh, wh, al, ar, lnc)


def _mlp_body(gm_ref, gma_ref, gd_ref, gda_ref, w1t_ref, w1b_ref, b1_ref,
              w2_ref, b2_ref, o_ref):
    hm = jnp.dot(gm_ref[...] + gma_ref[...], w1t_ref[...],
                 preferred_element_type=jnp.float32)
    hd = jnp.dot(gd_ref[...] + gda_ref[...], w1b_ref[...],
                 preferred_element_type=jnp.float32)
    h = hm + hd + b1_ref[...]
    h = jnp.where(h > 0, h, jnp.exp(h) - 1.0)
    r = jnp.dot(h, w2_ref[...], preferred_element_type=jnp.float32) + b2_ref[...]
    o_ref[...] = 1.0 / (1.0 + jnp.exp(-r))


def _mlp_head(gm, gma, gd, gda, w1, b1, w2, b2):
    b = gm.shape[0]
    tb = 512
    w1t = w1[:64]
    w1b = w1[64:]
    w2p = jnp.zeros((64, 128), jnp.float32).at[:, :1].set(w2)
    b2p = jnp.zeros((1, 128), jnp.float32).at[0, 0].set(b2[0])
    grid = (b // tb,)
    out = pl.pallas_call(
        _mlp_body,
        grid=grid,
        in_specs=[
            pl.BlockSpec((tb, 64), lambda i: (i, 0)),
            pl.BlockSpec((tb, 64), lambda i: (i, 0)),
            pl.BlockSpec((tb, 64), lambda i: (i, 0)),
            pl.BlockSpec((tb, 64), lambda i: (i, 0)),
            pl.BlockSpec((64, 64), lambda i: (0, 0)),
            pl.BlockSpec((64, 64), lambda i: (0, 0)),
            pl.BlockSpec((1, 64), lambda i: (0, 0)),
            pl.BlockSpec((64, 128), lambda i: (0, 0)),
            pl.BlockSpec((1, 128), lambda i: (0, 0)),
        ],
        out_specs=pl.BlockSpec((tb, 128), lambda i: (i, 0)),
        out_shape=jax.ShapeDtypeStruct((b, 128), jnp.float32),
        interpret=_interpret,
    )(gm, gma, gd, gda, w1t, w1b, b1.reshape(1, 64), w2p, b2p)
    return out[:, :1]


def _sc_build_counts(edge_index, n_nodes, n_chunk):
    src = edge_index[0].astype(jnp.int32)
    dst = edge_index[1].astype(jnp.int32)
    e = src.shape[0]
    info = plsc.get_sparse_core_info()
    nc, ns = info.num_cores, info.num_subcores
    ept = e // ns
    nj = ept // 128
    n_chunks = n_nodes // n_chunk
    rpc = n_chunk * n_nodes // 16
    zrows = rpc // ns
    dump = rpc

    zeros_sp = jnp.zeros((zrows, 16), jnp.float32)
    payload = (src[:, None] % 16 == lax.iota(jnp.int32, 16)[None, :]
               ).astype(jnp.float32)
    mesh = plsc.VectorSubcoreMesh(core_axis_name="c", subcore_axis_name="s")

    @functools.partial(
        pl.kernel, mesh=mesh,
        compiler_params=pltpu.CompilerParams(use_tc_tiling_on_sc=False),
        out_type=jax.ShapeDtypeStruct((n_nodes * n_nodes // 16, 16), jnp.float32),
        scratch_types=[
            pltpu.VMEM((ept,), jnp.int32),
            pltpu.VMEM((ept,), jnp.int32),
            pltpu.VMEM((ept, 16), jnp.float32),
            pltpu.VMEM((ept,), jnp.int32),
            pltpu.VMEM_SHARED((rpc + 8, 16), jnp.float32),
        ],
    )
    def k(src_h, dst_h, zsp_h, pay_h, out_h, src_v, dst_v, pay, ridx, acc):
        cid = lax.axis_index("c")
        sid = lax.axis_index("s")
        base = sid * ept
        pltpu.sync_copy(src_h.at[pl.ds(base, ept)], src_v)
        pltpu.sync_copy(dst_h.at[pl.ds(base, ept)], dst_v)
        pltpu.sync_copy(pay_h.at[pl.ds(base, ept)], pay)

        for c in range(n_chunks):
            @pl.when(cid == (c % nc))
            def _():
                pltpu.sync_copy(zsp_h, acc.at[pl.ds(sid * zrows, zrows)])
                plsc.subcore_barrier()

                def idx_body(g, carry):
                    sv = src_v[pl.ds(g * 16, 16)]
                    dv = dst_v[pl.ds(g * 16, 16)]
                    rel = dv - (c * n_chunk)
                    inb = jnp.logical_and(rel >= 0, rel < n_chunk)
                    row = rel * (n_nodes // 16) + lax.shift_right_logical(sv, 4)
                    row = jnp.where(inb, row, dump)
                    ridx[pl.ds(g * 16, 16)] = row
                    return carry

                lax.fori_loop(0, ept // 16, idx_body, 0)

                for r in range(ns):
                    @pl.when(sid == r)
                    def _():
                        pltpu.sync_copy(pay, acc.at[ridx], add=True)
                    plsc.subcore_barrier()
                pltpu.sync_copy(acc.at[pl.ds(sid * zrows, zrows)],
                                out_h.at[pl.ds(c * rpc + sid * zrows, zrows)])

    out = k(src, dst, zeros_sp, payload)
    return out.reshape(n_nodes, n_nodes)


def _sc_gather_embeddings(emb_m, emb_mm_ass, emb_d, emb_dd_ass, idx0, idx1):
    b = idx0.shape[0]
    info = plsc.get_sparse_core_info()
    nc, ns = info.num_cores, info.num_subcores
    bpw = b // (nc * ns)
    mesh = plsc.VectorSubcoreMesh(core_axis_name="c", subcore_axis_name="s")

    @functools.partial(
        pl.kernel, mesh=mesh,
        compiler_params=pltpu.CompilerParams(use_tc_tiling_on_sc=False),
        out_type=[jax.ShapeDtypeStruct((b, 64), jnp.float32)] * 4,
        scratch_types=[
            pltpu.VMEM((bpw,), jnp.int32),
            pltpu.VMEM((bpw,), jnp.int32),
            pltpu.VMEM((bpw, 64), jnp.float32),
            pltpu.SemaphoreType.DMA,
        ],
    )
    def k(em, ema, ed, eda, i0, i1, o0, o1, o2, o3, iv0, iv1, rows, sem):
        wid = lax.axis_index("s") * nc + lax.axis_index("c")
        base = wid * bpw
        pltpu.sync_copy(i0.at[pl.ds(base, bpw)], iv0)
        pltpu.sync_copy(i1.at[pl.ds(base, bpw)], iv1)
        for table, iv, out in ((em, iv0, o0), (ema, iv0, o1),
                               (ed, iv1, o2), (eda, iv1, o3)):
            pltpu.async_copy(table.at[iv], rows, sem).wait()
            pltpu.sync_copy(rows, out.at[pl.ds(base, bpw)])

    return k(emb_m, emb_mm_ass, emb_d, emb_dd_ass, idx0, idx1)


def _gt_block(x, counts, layers, extra_res):
    n, d = x.shape
    scale = 1.0 / ((d // GT_HEAD) ** 0.5)
    for li, lp in enumerate(layers):
        wqkv = jnp.concatenate([lp['Wq'] * scale, lp['Wk'], lp['Wv']], axis=1)
        qkv = _matmul(x, wqkv)
        agg = _gt_attention(qkv, counts, n, d)
        res = (x,) if (li < len(layers) - 1 or extra_res is None) else (x, extra_res)
        x = _matmul(agg, lp['Wo'], residuals=res)
    return x


def _gat_block(x, counts, p):
    n = x.shape[0]
    for lp in p['layers']:
        h = lp['al'].shape[-1]
        wh = _matmul(x, lp['W'])
        x = _gat_attention(wh, lp['al'], lp['ar'], counts, n, h)
    return _matmul(x, p['Wout'])


def kernel(microe, disease, params, mm_graph, dd_graph, md_graph, samples, epoch):
    c_mm = _ln_counts(_sc_build_counts(mm_graph, MIC, 512), MIC)
    c_dd = _ln_counts(_sc_build_counts(dd_graph, DIS, 512), DIS)
    c_md = _ln_counts(_sc_build_counts(md_graph, MIC + DIS, 512), MIC + DIS)

    xm = _gt_block(microe, c_mm, params['gt_m'], extra_res=microe)
    xd = _gt_block(disease, c_dd, params['gt_d'], extra_res=disease)

    emb_m = _gat_block(xm, c_mm, params['gat_m'])
    emb_d = _gat_block(xd, c_dd, params['gat_d'])

    xmd_top = _matmul(microe, 2.0 * params['lin_m'])
    xmd_bot = _matmul(disease, 2.0 * params['lin_d'])
    xmd = jnp.concatenate([xmd_top, xmd_bot], axis=0)
    emb_md = _gat_block(xmd, c_md, params['gat_md'])
    emb_mm_ass = emb_md[:MIC]
    emb_dd_ass = emb_md[MIC:]

    idx0 = samples[:, 0].astype(jnp.int32)
    idx1 = samples[:, 1].astype(jnp.int32)
    gm, gma, gd, gda = _sc_gather_embeddings(emb_m, emb_mm_ass, emb_d,
                                             emb_dd_ass, idx0, idx1)

    mlp = params['mlp']
    result = _mlp_head(gm, gma, gd, gda, mlp['W1'], mlp['b1'], mlp['W2'], mlp['b2'])
    return (result, emb_m, emb_mm_ass, emb_d, emb_dd_ass)


# speedup vs baseline: 1.0254x; 1.0003x over previous
import functools

import jax
import jax.numpy as jnp
from jax import lax
from jax.experimental import pallas as pl
from jax.experimental.pallas import tpu as pltpu
from jax.experimental.pallas import tpu_sc as plsc


MIC = 2048
DIS = 1024
GT_HEAD = 4
GAT_HEADS = 10


def _pick_tile(n, cands):
    for c in cands:
        if n % c == 0:
            return c
    return n


def _mm_body(nres, x_ref, w_ref, *refs):
    out_ref = refs[-1]
    acc = jnp.dot(x_ref[...], w_ref[...], preferred_element_type=jnp.float32)
    for r in refs[:nres]:
        acc = acc + r[...]
    out_ref[...] = acc


def _matmul(x, w, residuals=()):
    M, K = x.shape
    _, N = w.shape
    tm = _pick_tile(M, (256, 128, 64))
    tn = _pick_tile(N, (512, 256, 128, 64))
    grid = (N // tn, M // tm)
    in_specs = [
        pl.BlockSpec((tm, K), lambda j, i: (i, 0)),
        pl.BlockSpec((K, tn), lambda j, i: (0, j)),
    ] + [pl.BlockSpec((tm, tn), lambda j, i: (i, j)) for _ in residuals]
    return pl.pallas_call(
        functools.partial(_mm_body, len(residuals)),
        grid=grid,
        in_specs=in_specs,
        out_specs=pl.BlockSpec((tm, tn), lambda j, i: (i, j)),
        out_shape=jax.ShapeDtypeStruct((M, N), jnp.float32),
    )(x, w, *residuals)


def _lnc_body(c_ref, o_ref):
    c = c_ref[...]
    o_ref[...] = jnp.where(c > 0.0, jnp.log(c), -jnp.inf)


def _ln_counts(counts, n):
    tr = 256
    return pl.pallas_call(
        _lnc_body,
        grid=(n // tr,),
        in_specs=[pl.BlockSpec((tr, n), lambda i: (i, 0))],
        out_specs=pl.BlockSpec((tr, n), lambda i: (i, 0)),
        out_shape=jax.ShapeDtypeStruct((n, n), jnp.float32),
    )(counts)


def _masked_softmax_parts(s):
    m = jnp.max(s, axis=1, keepdims=True)
    mf = jnp.maximum(m, -1e30)
    e = jnp.exp(s - mf)
    return e, 1.0 / (jnp.sum(e, axis=1, keepdims=True) + 1e-9)


def _gt_attn_body(dh, q_ref, k_ref, v_ref, lnc_ref, o_ref):
    lnc = lnc_ref[...]
    outs = []
    for h in range(GT_HEAD):
        sl = slice(h * dh, (h + 1) * dh)
        s = lax.dot_general(q_ref[:, sl], k_ref[:, sl], (((1,), (1,)), ((), ())),
                            preferred_element_type=jnp.float32) + lnc
        e, inv = _masked_softmax_parts(s)
        outs.append(jnp.dot(e, v_ref[:, sl],
                            preferred_element_type=jnp.float32) * inv)
    o_ref[...] = jnp.concatenate(outs, axis=1)


def _gt_attention(qkv, lnc, n, d):
    dh = d // GT_HEAD
    td = 256
    grid = (n // td,)
    return pl.pallas_call(
        functools.partial(_gt_attn_body, dh),
        grid=grid,
        in_specs=[
            pl.BlockSpec((td, d), lambda i: (i, 0)),
            pl.BlockSpec((n, d), lambda i: (0, 1)),
            pl.BlockSpec((n, d), lambda i: (0, 2)),
            pl.BlockSpec((td, n), lambda i: (i, 0)),
        ],
        out_specs=pl.BlockSpec((td, d), lambda i: (i, 0)),
        out_shape=jax.ShapeDtypeStruct((n, d), jnp.float32),
    )(qkv, qkv, qkv, lnc)


def _gat_attn_body(h, whd_ref, whs_ref, al_ref, ar_ref, lnc_ref, o_ref):
    whd = whd_ref[...]
    whs = whs_ref[...]
    al = al_ref[...]
    ar = ar_ref[...]
    lnc = lnc_ref[...]
    outs = []
    for t in range(GAT_HEADS):
        sl = slice(t * h, (t + 1) * h)
        whd_t = whd[:, sl]
        whs_t = whs[:, sl]
        ed = lax.dot_general(whd_t, al[t:t + 1, :], (((1,), (1,)), ((), ())),
                             preferred_element_type=jnp.float32)
        es = lax.dot_general(ar[t:t + 1, :], whs_t, (((1,), (1,)), ((), ())),
                             preferred_element_type=jnp.float32)
        s = ed + es
        s = jnp.where(s >= 0, s, 0.2 * s) + lnc
        e, inv = _masked_softmax_parts(s)
        out = jnp.dot(e, whs_t, preferred_element_type=jnp.float32) * inv
        outs.append(jnp.where(out > 0, out, jnp.exp(out) - 1.0))
    o_ref[...] = jnp.concatenate(outs, axis=1)


def _gat_attention(wh, al, ar, lnc, n, h):
    td = 256
    grid = (n // td,)
    return pl.pallas_call(
        functools.partial(_gat_attn_body, h),
        grid=grid,
        in_specs=[
            pl.BlockSpec((td, GAT_HEADS * h), lambda i: (i, 0)),
            pl.BlockSpec((n, GAT_HEADS * h), lambda i: (0, 0)),
            pl.BlockSpec((GAT_HEADS, h), lambda i: (0, 0)),
            pl.BlockSpec((GAT_HEADS, h), lambda i: (0, 0)),
            pl.BlockSpec((td, n), lambda i: (i, 0)),
        ],
        out_specs=pl.BlockSpec((td, GAT_HEADS * h), lambda i: (i, 0)),
        out_shape=jax.ShapeDtypeStruct((n, GAT_HEADS * h), jnp.float32),
    )(wh, wh, al, ar, lnc)


def _mlp_body(gm_ref, gma_ref, gd_ref, gda_ref, w1t_ref, w1b_ref, b1_ref,
              w2_ref, b2_ref, o_ref):
    hm = jnp.dot(gm_ref[...] + gma_ref[...], w1t_ref[...],
                 preferred_element_type=jnp.float32)
    hd = jnp.dot(gd_ref[...] + gda_ref[...], w1b_ref[...],
                 preferred_element_type=jnp.float32)
    h = hm + hd + b1_ref[...]
    h = jnp.where(h > 0, h, jnp.exp(h) - 1.0)
    r = jnp.dot(h, w2_ref[...], preferred_element_type=jnp.float32) + b2_ref[...]
    o_ref[...] = 1.0 / (1.0 + jnp.exp(-r))


def _mlp_head(gm, gma, gd, gda, w1, b1, w2, b2):
    b = gm.shape[0]
    tb = 512
    w1t = w1[:64]
    w1b = w1[64:]
    w2p = jnp.zeros((64, 128), jnp.float32).at[:, :1].set(w2)
    b2p = jnp.zeros((1, 128), jnp.float32).at[0, 0].set(b2[0])
    grid = (b // tb,)
    out = pl.pallas_call(
        _mlp_body,
        grid=grid,
        in_specs=[
            pl.BlockSpec((tb, 64), lambda i: (i, 0)),
            pl.BlockSpec((tb, 64), lambda i: (i, 0)),
            pl.BlockSpec((tb, 64), lambda i: (i, 0)),
            pl.BlockSpec((tb, 64), lambda i: (i, 0)),
            pl.BlockSpec((64, 64), lambda i: (0, 0)),
            pl.BlockSpec((64, 64), lambda i: (0, 0)),
            pl.BlockSpec((1, 64), lambda i: (0, 0)),
            pl.BlockSpec((64, 128), lambda i: (0, 0)),
            pl.BlockSpec((1, 128), lambda i: (0, 0)),
        ],
        out_specs=pl.BlockSpec((tb, 128), lambda i: (i, 0)),
        out_shape=jax.ShapeDtypeStruct((b, 128), jnp.float32),
    )(gm, gma, gd, gda, w1t, w1b, b1.reshape(1, 64), w2p, b2p)
    return out[:, :1]


def _sc_build_counts(edge_index, n_nodes, n_chunk):
    src = edge_index[0].astype(jnp.int32)
    dst = edge_index[1].astype(jnp.int32)
    e = src.shape[0]
    info = plsc.get_sparse_core_info()
    nc, ns = info.num_cores, info.num_subcores
    ept = e // ns
    nj = ept // 128
    n_chunks = n_nodes // n_chunk
    rpc = n_chunk * n_nodes // 16
    zrows = rpc // ns
    dump = rpc

    zeros_sp = jnp.zeros((zrows, 16), jnp.float32)
    payload = (src[:, None] % 16 == lax.iota(jnp.int32, 16)[None, :]
               ).astype(jnp.float32)
    mesh = plsc.VectorSubcoreMesh(core_axis_name="c", subcore_axis_name="s")

    @functools.partial(
        pl.kernel, mesh=mesh,
        compiler_params=pltpu.CompilerParams(use_tc_tiling_on_sc=False),
        out_type=jax.ShapeDtypeStruct((n_nodes * n_nodes // 16, 16), jnp.float32),
        scratch_types=[
            pltpu.VMEM((ept,), jnp.int32),
            pltpu.VMEM((ept,), jnp.int32),
            pltpu.VMEM((ept, 16), jnp.float32),
            pltpu.VMEM((ept,), jnp.int32),
            pltpu.VMEM_SHARED((rpc + 8, 16), jnp.float32),
        ],
    )
    def k(src_h, dst_h, zsp_h, pay_h, out_h, src_v, dst_v, pay, ridx, acc):
        cid = lax.axis_index("c")
        sid = lax.axis_index("s")
        base = sid * ept
        pltpu.sync_copy(src_h.at[pl.ds(base, ept)], src_v)
        pltpu.sync_copy(dst_h.at[pl.ds(base, ept)], dst_v)
        pltpu.sync_copy(pay_h.at[pl.ds(base, ept)], pay)

        for c in range(n_chunks):
            @pl.when(cid == (c % nc))
            def _():
                pltpu.sync_copy(zsp_h, acc.at[pl.ds(sid * zrows, zrows)])
                plsc.subcore_barrier()

                def idx_body(g, carry):
                    sv = src_v[pl.ds(g * 16, 16)]
                    dv = dst_v[pl.ds(g * 16, 16)]
                    rel = dv - (c * n_chunk)
                    inb = jnp.logical_and(rel >= 0, rel < n_chunk)
                    row = rel * (n_nodes // 16) + lax.shift_right_logical(sv, 4)
                    row = jnp.where(inb, row, dump)
                    ridx[pl.ds(g * 16, 16)] = row
                    return carry

                lax.fori_loop(0, ept // 16, idx_body, 0)

                for r in range(ns):
                    @pl.when(sid == r)
                    def _():
                        pltpu.sync_copy(pay, acc.at[ridx], add=True)
                    plsc.subcore_barrier()
                pltpu.sync_copy(acc.at[pl.ds(sid * zrows, zrows)],
                                out_h.at[pl.ds(c * rpc + sid * zrows, zrows)])

    out = k(src, dst, zeros_sp, payload)
    return out.reshape(n_nodes, n_nodes)


def _sc_gather_embeddings(emb_m, emb_mm_ass, emb_d, emb_dd_ass, idx0, idx1):
    b = idx0.shape[0]
    info = plsc.get_sparse_core_info()
    nc, ns = info.num_cores, info.num_subcores
    bpw = b // (nc * ns)
    mesh = plsc.VectorSubcoreMesh(core_axis_name="c", subcore_axis_name="s")

    @functools.partial(
        pl.kernel, mesh=mesh,
        compiler_params=pltpu.CompilerParams(use_tc_tiling_on_sc=False),
        out_type=[jax.ShapeDtypeStruct((b, 64), jnp.float32)] * 4,
        scratch_types=[
            pltpu.VMEM((bpw,), jnp.int32),
            pltpu.VMEM((bpw,), jnp.int32),
            pltpu.VMEM((bpw, 64), jnp.float32),
            pltpu.SemaphoreType.DMA,
        ],
    )
    def k(em, ema, ed, eda, i0, i1, o0, o1, o2, o3, iv0, iv1, rows, sem):
        wid = lax.axis_index("s") * nc + lax.axis_index("c")
        base = wid * bpw
        pltpu.sync_copy(i0.at[pl.ds(base, bpw)], iv0)
        pltpu.sync_copy(i1.at[pl.ds(base, bpw)], iv1)
        for table, iv, out in ((em, iv0, o0), (ema, iv0, o1),
                               (ed, iv1, o2), (eda, iv1, o3)):
            pltpu.async_copy(table.at[iv], rows, sem).wait()
            pltpu.sync_copy(rows, out.at[pl.ds(base, bpw)])

    return k(emb_m, emb_mm_ass, emb_d, emb_dd_ass, idx0, idx1)


def _gt_block(x, counts, layers, extra_res):
    n, d = x.shape
    scale = 1.0 / ((d // GT_HEAD) ** 0.5)
    for li, lp in enumerate(layers):
        wqkv = jnp.concatenate([lp['Wq'] * scale, lp['Wk'], lp['Wv']], axis=1)
        qkv = _matmul(x, wqkv)
        agg = _gt_attention(qkv, counts, n, d)
        res = (x,) if (li < len(layers) - 1 or extra_res is None) else (x, extra_res)
        x = _matmul(agg, lp['Wo'], residuals=res)
    return x


def _gat_block(x, counts, p):
    n = x.shape[0]
    for lp in p['layers']:
        h = lp['al'].shape[-1]
        wh = _matmul(x, lp['W'])
        x = _gat_attention(wh, lp['al'], lp['ar'], counts, n, h)
    return _matmul(x, p['Wout'])


def kernel(microe, disease, params, mm_graph, dd_graph, md_graph, samples, epoch):
    c_mm = _ln_counts(_sc_build_counts(mm_graph, MIC, 512), MIC)
    c_dd = _ln_counts(_sc_build_counts(dd_graph, DIS, 512), DIS)
    c_md = _ln_counts(_sc_build_counts(md_graph, MIC + DIS, 512), MIC + DIS)

    xm = _gt_block(microe, c_mm, params['gt_m'], extra_res=microe)
    xd = _gt_block(disease, c_dd, params['gt_d'], extra_res=disease)

    emb_m = _gat_block(xm, c_mm, params['gat_m'])
    emb_d = _gat_block(xd, c_dd, params['gat_d'])

    xmd_top = _matmul(microe, 2.0 * params['lin_m'])
    xmd_bot = _matmul(disease, 2.0 * params['lin_d'])
    xmd = jnp.concatenate([xmd_top, xmd_bot], axis=0)
    emb_md = _gat_block(xmd, c_md, params['gat_md'])
    emb_mm_ass = emb_md[:MIC]
    emb_dd_ass = emb_md[MIC:]

    idx0 = samples[:, 0].astype(jnp.int32)
    idx1 = samples[:, 1].astype(jnp.int32)
    gm, gma, gd, gda = _sc_gather_embeddings(emb_m, emb_mm_ass, emb_d,
                                             emb_dd_ass, idx0, idx1)

    mlp = params['mlp']
    result = _mlp_head(gm, gma, gd, gda, mlp['W1'], mlp['b1'], mlp['W2'], mlp['b2'])
    return (result, emb_m, emb_mm_ass, emb_d, emb_dd_ass)
